# baseline (speedup 1.0000x reference)
import functools

import jax
import jax.numpy as jnp
from jax import lax
from jax.experimental import pallas as pl
from jax.experimental.pallas import tpu as pltpu
from jax.experimental.pallas import tpu_sc as plsc

_INFO = plsc.get_sparse_core_info()
_NC = _INFO.num_cores
_NS = _INFO.num_subcores
_NW = _NC * _NS

_CHUNK = 16
_NBUF = 2
_TC_ROWS = 4096
_TC_RB = 8


def _make_sc_kernel(n_rows, hidden):
    rows_per_w = n_rows // _NW
    n_chunks = rows_per_w // _CHUNK
    vecs_per_row = hidden // 16
    mesh = plsc.VectorSubcoreMesh(core_axis_name="c", subcore_axis_name="s")

    @functools.partial(
        pl.kernel,
        mesh=mesh,
        out_type=jax.ShapeDtypeStruct((n_rows, hidden), jnp.float32),
        scratch_types=[
            pltpu.VMEM((rows_per_w,), jnp.int32),
            pltpu.VMEM((rows_per_w,), jnp.int32),
            pltpu.VMEM((_NBUF, _CHUNK, hidden), jnp.float32),
            pltpu.VMEM((_NBUF, _CHUNK, hidden), jnp.float32),
            pltpu.VMEM((_NBUF, _CHUNK, hidden), jnp.float32),
        ]
        + [pltpu.SemaphoreType.DMA] * (3 * _NBUF),
    )
    def k(widx_hbm, pidx_hbm, word_hbm, pos_hbm, out_hbm,
          widx_v, pidx_v, wbuf, pbuf, obuf, *sems):
        sem_w = sems[0:_NBUF]
        sem_p = sems[_NBUF:2 * _NBUF]
        sem_o = sems[2 * _NBUF:3 * _NBUF]
        wid = lax.axis_index("s") * _NC + lax.axis_index("c")
        base = wid * rows_per_w

        pltpu.sync_copy(widx_hbm.at[pl.ds(base, rows_per_w)], widx_v)
        pltpu.sync_copy(pidx_hbm.at[pl.ds(base, rows_per_w)], pidx_v)

        def start_gathers(c, b):
            idx = pl.ds(c * _CHUNK, _CHUNK)
            pltpu.async_copy(word_hbm.at[widx_v.at[idx]], wbuf.at[b], sem_w[b])
            pltpu.async_copy(pos_hbm.at[pidx_v.at[idx]], pbuf.at[b], sem_p[b])

        def wait_gathers(c, b):
            idx = pl.ds(c * _CHUNK, _CHUNK)
            pltpu.make_async_copy(word_hbm.at[widx_v.at[idx]], wbuf.at[b], sem_w[b]).wait()
            pltpu.make_async_copy(pos_hbm.at[pidx_v.at[idx]], pbuf.at[b], sem_p[b]).wait()

        def out_slice(c):
            return out_hbm.at[pl.ds(base + c * _CHUNK, _CHUNK)]

        for b in range(_NBUF):
            start_gathers(b, b)

        def chunk_group(g, _):
            for b in range(_NBUF):
                c = g * _NBUF + b
                wait_gathers(c, b)

                @pl.when(c >= _NBUF)
                def _():
                    pltpu.make_async_copy(obuf.at[b], out_slice(c - _NBUF), sem_o[b]).wait()

                def add_body(i, _):
                    r = i // vecs_per_row
                    j = (i % vecs_per_row) * 16
                    obuf[b, r, pl.ds(j, 16)] = (
                        wbuf[b, r, pl.ds(j, 16)] + pbuf[b, r, pl.ds(j, 16)]
                    )
                    return 0

                lax.fori_loop(0, _CHUNK * vecs_per_row, add_body, 0, unroll=8)

                pltpu.async_copy(obuf.at[b], out_slice(c), sem_o[b])

                @pl.when(c + _NBUF < n_chunks)
                def _():
                    start_gathers(c + _NBUF, b)
            return 0

        lax.fori_loop(0, n_chunks // _NBUF, chunk_group, 0)

        for b in range(_NBUF):
            c = n_chunks - _NBUF + b
            pltpu.make_async_copy(obuf.at[b], out_slice(c), sem_o[b]).wait()

    return k


def _make_tc_kernel(n_rows, hidden):
    grid = (n_rows // _TC_RB,)
    sub = hidden // 8

    def body(widx_s, pidx_s, *refs):
        out_ref = refs[-1]
        for r in range(_TC_RB):
            out_ref[r, :, :] = refs[r][0, :, :] + refs[_TC_RB + r][0, :, :]

    def w_map(r):
        return lambda i, widx, pidx: (widx[i * _TC_RB + r], 0, 0)

    def p_map(r):
        return lambda i, widx, pidx: (pidx[i * _TC_RB + r], 0, 0)

    in_specs = [pl.BlockSpec((1, 8, sub), w_map(r)) for r in range(_TC_RB)]
    in_specs += [pl.BlockSpec((1, 8, sub), p_map(r)) for r in range(_TC_RB)]
    gs = pltpu.PrefetchScalarGridSpec(
        num_scalar_prefetch=2,
        grid=grid,
        in_specs=in_specs,
        out_specs=pl.BlockSpec((_TC_RB, 8, sub), lambda i, widx, pidx: (i, 0, 0)),
    )
    return pl.pallas_call(
        body,
        grid_spec=gs,
        out_shape=jax.ShapeDtypeStruct((n_rows, 8, sub), jnp.float32),
    )


def kernel(input_ids, position_ids, word_embeddings, position_embeddings):
    batch, seq = input_ids.shape
    hidden = word_embeddings.shape[1]
    n_rows = batch * seq

    widx = jnp.transpose(input_ids, (1, 0)).reshape(n_rows).astype(jnp.int32)
    pidx = jnp.transpose(position_ids, (1, 0)).reshape(n_rows).astype(jnp.int32)

    n_sc = n_rows - _TC_ROWS
    sc = _make_sc_kernel(n_sc, hidden)
    tc = _make_tc_kernel(_TC_ROWS, hidden)
    flat_sc = sc(widx[:n_sc], pidx[:n_sc], word_embeddings, position_embeddings)
    w3 = word_embeddings.reshape(-1, 8, hidden // 8)
    p3 = position_embeddings.reshape(-1, 8, hidden // 8)
    tables = [w3] * _TC_RB + [p3] * _TC_RB
    flat_tc = tc(widx[n_sc:], pidx[n_sc:], *tables).reshape(_TC_ROWS, hidden)
    flat = jnp.concatenate([flat_sc, flat_tc], axis=0)
    return flat.reshape(seq, batch, hidden)


# speedup vs baseline: 3.8063x; 3.8063x over previous
import functools

import jax
import jax.numpy as jnp
from jax import lax
from jax.experimental import pallas as pl
from jax.experimental.pallas import tpu as pltpu
from jax.experimental.pallas import tpu_sc as plsc

_INFO = plsc.get_sparse_core_info()
_NC = _INFO.num_cores
_NS = _INFO.num_subcores
_NW = _NC * _NS

_CHUNK = 16
_NBUF = 2


def _make_sc_kernel(n_rows, hidden):
    rows_per_w = n_rows // _NW
    n_chunks = rows_per_w // _CHUNK
    vecs_per_row = hidden // 16
    mesh = plsc.VectorSubcoreMesh(core_axis_name="c", subcore_axis_name="s")

    @functools.partial(
        pl.kernel,
        mesh=mesh,
        out_type=jax.ShapeDtypeStruct((n_rows, hidden), jnp.float32),
        scratch_types=[
            pltpu.VMEM((rows_per_w,), jnp.int32),
            pltpu.VMEM((rows_per_w,), jnp.int32),
            pltpu.VMEM((_NBUF, _CHUNK, hidden), jnp.float32),
            pltpu.VMEM((_NBUF, _CHUNK, hidden), jnp.float32),
            pltpu.VMEM((_NBUF, _CHUNK, hidden), jnp.float32),
        ]
        + [pltpu.SemaphoreType.DMA] * (3 * _NBUF),
    )
    def k(widx_hbm, pidx_hbm, word_hbm, pos_hbm, out_hbm,
          widx_v, pidx_v, wbuf, pbuf, obuf, *sems):
        sem_w = sems[0:_NBUF]
        sem_p = sems[_NBUF:2 * _NBUF]
        sem_o = sems[2 * _NBUF:3 * _NBUF]
        wid = lax.axis_index("s") * _NC + lax.axis_index("c")
        base = wid * rows_per_w

        pltpu.sync_copy(widx_hbm.at[pl.ds(base, rows_per_w)], widx_v)
        pltpu.sync_copy(pidx_hbm.at[pl.ds(base, rows_per_w)], pidx_v)

        def start_gathers(c, b):
            idx = pl.ds(c * _CHUNK, _CHUNK)
            pltpu.async_copy(word_hbm.at[widx_v.at[idx]], wbuf.at[b], sem_w[b])
            pltpu.async_copy(pos_hbm.at[pidx_v.at[idx]], pbuf.at[b], sem_p[b])

        def wait_gathers(c, b):
            idx = pl.ds(c * _CHUNK, _CHUNK)
            pltpu.make_async_copy(word_hbm.at[widx_v.at[idx]], wbuf.at[b], sem_w[b]).wait()
            pltpu.make_async_copy(pos_hbm.at[pidx_v.at[idx]], pbuf.at[b], sem_p[b]).wait()

        def out_slice(c):
            return out_hbm.at[pl.ds(base + c * _CHUNK, _CHUNK)]

        for b in range(_NBUF):
            start_gathers(b, b)

        def chunk_group(g, _):
            for b in range(_NBUF):
                c = g * _NBUF + b
                wait_gathers(c, b)

                @pl.when(c >= _NBUF)
                def _():
                    pltpu.make_async_copy(obuf.at[b], out_slice(c - _NBUF), sem_o[b]).wait()

                def add_body(i, _):
                    r = i // vecs_per_row
                    j = (i % vecs_per_row) * 16
                    obuf[b, r, pl.ds(j, 16)] = (
                        wbuf[b, r, pl.ds(j, 16)] + pbuf[b, r, pl.ds(j, 16)]
                    )
                    return 0

                lax.fori_loop(0, _CHUNK * vecs_per_row, add_body, 0, unroll=8)

                pltpu.async_copy(obuf.at[b], out_slice(c), sem_o[b])

                @pl.when(c + _NBUF < n_chunks)
                def _():
                    start_gathers(c + _NBUF, b)
            return 0

        lax.fori_loop(0, n_chunks // _NBUF, chunk_group, 0)

        for b in range(_NBUF):
            c = n_chunks - _NBUF + b
            pltpu.make_async_copy(obuf.at[b], out_slice(c), sem_o[b]).wait()

    return k


def kernel(input_ids, position_ids, word_embeddings, position_embeddings):
    batch, seq = input_ids.shape
    hidden = word_embeddings.shape[1]
    n_rows = batch * seq

    widx = jnp.transpose(input_ids, (1, 0)).reshape(n_rows).astype(jnp.int32)
    pidx = jnp.transpose(position_ids, (1, 0)).reshape(n_rows).astype(jnp.int32)

    k = _make_sc_kernel(n_rows, hidden)
    flat = k(widx, pidx, word_embeddings, position_embeddings)
    return flat.reshape(seq, batch, hidden)
